# Initial kernel scaffold; baseline (speedup 1.0000x reference)
import functools

import jax
import jax.numpy as jnp
from jax import lax
from jax.experimental import pallas as pl
from jax.experimental.pallas import tpu as pltpu
from jax.experimental.pallas import tpu_sc as plsc

NC = 2
NS = 16
NW = NC * NS
C = 128


def _sc_body(ctx, x_hbm, src_hbm, dst_hbm, sums_hbm, cnts_hbm,
             src_v, dst_v, rows_v, ones_v, zrows_v, zcnt_v,
             acc_sh, cnt_sh, sem):
    n_pad, rpt, k_chunks, epw = ctx
    cid = lax.axis_index("c")
    sid = lax.axis_index("s")
    wid = sid * NC + cid

    zeros16 = jnp.zeros((16,), jnp.float32)
    ones16 = jnp.ones((16,), jnp.float32)

    @pl.loop(0, C * 8)
    def _(t):
        zrows_v[t // 8, pl.ds(16 * (t % 8), 16)] = zeros16

    @pl.loop(0, rpt)
    def _(i):
        zcnt_v[i, :] = zeros16

    @pl.loop(0, C)
    def _(i):
        ones_v[i, :] = ones16

    row0 = sid * rpt

    @pl.loop(0, rpt // C)
    def _(j):
        pltpu.sync_copy(zrows_v, acc_sh.at[pl.ds(row0 + j * C, C)])

    pltpu.sync_copy(zcnt_v, cnt_sh.at[pl.ds(row0, rpt)])

    plsc.subcore_barrier()

    base_w = wid * epw

    @pl.loop(0, k_chunks)
    def _(k):
        base = base_w + k * C
        pltpu.sync_copy(src_hbm.at[pl.ds(base, C)], src_v)
        pltpu.sync_copy(dst_hbm.at[pl.ds(base, C)], dst_v)
        pltpu.async_copy(x_hbm.at[src_v], rows_v, sem).wait()
        pltpu.sync_copy(rows_v, acc_sh.at[dst_v], add=True)
        pltpu.sync_copy(ones_v, cnt_sh.at[dst_v], add=True)

    plsc.subcore_barrier()

    pltpu.sync_copy(acc_sh.at[pl.ds(row0, rpt)],
                    sums_hbm.at[cid, pl.ds(row0, rpt)])
    pltpu.sync_copy(cnt_sh.at[pl.ds(row0, rpt)],
                    cnts_hbm.at[cid, pl.ds(row0, rpt)])


def _segment_sums(x, src, dst, n_pad):
    e_pad = src.shape[0]
    epw = e_pad // NW
    k_chunks = epw // C
    rpt = n_pad // NS
    d = x.shape[1]
    mesh = plsc.VectorSubcoreMesh(core_axis_name="c", subcore_axis_name="s")
    body = functools.partial(_sc_body, (n_pad, rpt, k_chunks, epw))
    return pl.kernel(
        body,
        out_type=(
            jax.ShapeDtypeStruct((NC, n_pad, d), jnp.float32),
            jax.ShapeDtypeStruct((NC, n_pad, 16), jnp.float32),
        ),
        mesh=mesh,
        scratch_types=[
            pltpu.VMEM((C,), jnp.int32),
            pltpu.VMEM((C,), jnp.int32),
            pltpu.VMEM((C, d), jnp.float32),
            pltpu.VMEM((C, 16), jnp.float32),
            pltpu.VMEM((C, d), jnp.float32),
            pltpu.VMEM((rpt, 16), jnp.float32),
            pltpu.VMEM_SHARED((n_pad, d), jnp.float32),
            pltpu.VMEM_SHARED((n_pad, 16), jnp.float32),
            pltpu.SemaphoreType.DMA,
        ],
    )(x, src, dst)


def _tc_body(s_ref, c_ref, w_ref, b_ref, o_ref):
    s = s_ref[0] + s_ref[1]
    c = c_ref[0, :, 0:1] + c_ref[1, :, 0:1]
    mean = s / jnp.maximum(c, 1.0)
    h = lax.dot_general(mean, w_ref[...], (((1,), (1,)), ((), ())),
                        preferred_element_type=jnp.float32)
    out = h + b_ref[...] * jnp.minimum(c, 1.0)
    o_ref[...] = jnp.maximum(out, 0.0)


def _finish(sums, cnts, W, b, n_pad, rows_blk):
    d_in = W.shape[1]
    d_out = W.shape[0]
    grid = (n_pad // rows_blk,)
    return pl.pallas_call(
        _tc_body,
        grid=grid,
        in_specs=[
            pl.BlockSpec((NC, rows_blk, d_in), lambda i: (0, i, 0)),
            pl.BlockSpec((NC, rows_blk, 16), lambda i: (0, i, 0)),
            pl.BlockSpec((d_out, d_in), lambda i: (0, 0)),
            pl.BlockSpec((1, d_out), lambda i: (0, 0)),
        ],
        out_specs=pl.BlockSpec((rows_blk, d_out), lambda i: (i, 0)),
        out_shape=jax.ShapeDtypeStruct((n_pad, d_out), jnp.float32),
    )(sums, cnts, W, b.reshape(1, d_out))


def kernel(x, edge_index, W, b):
    n = x.shape[0]
    e = edge_index.shape[1]

    n_pad = ((n + 1) + NS * C - 1) // (NS * C) * (NS * C)
    e_pad = (e + NW * C - 1) // (NW * C) * (NW * C)

    src = edge_index[0].astype(jnp.int32)
    dst = edge_index[1].astype(jnp.int32)
    pad = e_pad - e
    if pad:
        src = jnp.concatenate([src, jnp.zeros((pad,), jnp.int32)])
        dst = jnp.concatenate([dst, jnp.full((pad,), n, jnp.int32)])

    sums, cnts = _segment_sums(x, src, dst, n_pad)
    out = _finish(sums, cnts, W, b, n_pad, rows_blk=1024)
    return out[:n]


# speedup vs baseline: 4.5621x; 4.5621x over previous
import functools

import jax
import jax.numpy as jnp
from jax import lax
from jax.experimental import pallas as pl
from jax.experimental.pallas import tpu as pltpu
from jax.experimental.pallas import tpu_sc as plsc

NC = 2
NS = 16
NW = NC * NS
C = 128


def _sc_body(ctx, x_hbm, src_hbm, dst_hbm, sums_hbm, cnts_hbm,
             src_v, dst_v, rows_v, ones_v,
             acc_sh, cnt_sh, sem):
    n_pad, rpt, k_chunks, epw = ctx
    cid = lax.axis_index("c")
    sid = lax.axis_index("s")
    wid = sid * NC + cid

    zeros16 = jnp.zeros((16,), jnp.float32)
    ones16 = jnp.ones((16,), jnp.float32)

    @pl.loop(0, C * 8)
    def _(t):
        rows_v[t // 8, pl.ds(16 * (t % 8), 16)] = zeros16

    @pl.loop(0, C)
    def _(i):
        ones_v[i, :] = zeros16

    row0 = sid * rpt

    @pl.loop(0, rpt // C)
    def _(j):
        pltpu.sync_copy(rows_v, acc_sh.at[pl.ds(row0 + j * C, C)])
        pltpu.sync_copy(ones_v, cnt_sh.at[pl.ds(row0 + j * C, C)])

    @pl.loop(0, C)
    def _(i):
        ones_v[i, :] = ones16

    plsc.subcore_barrier()

    base_w = wid * epw

    @pl.loop(0, k_chunks)
    def _(k):
        base = base_w + k * C
        pltpu.sync_copy(src_hbm.at[pl.ds(base, C)], src_v)
        pltpu.sync_copy(dst_hbm.at[pl.ds(base, C)], dst_v)
        pltpu.async_copy(x_hbm.at[src_v], rows_v, sem).wait()
        pltpu.sync_copy(rows_v, acc_sh.at[dst_v], add=True)
        pltpu.sync_copy(ones_v, cnt_sh.at[dst_v], add=True)

    plsc.subcore_barrier()

    pltpu.sync_copy(acc_sh.at[pl.ds(row0, rpt)],
                    sums_hbm.at[cid, pl.ds(row0, rpt)])
    pltpu.sync_copy(cnt_sh.at[pl.ds(row0, rpt)],
                    cnts_hbm.at[cid, pl.ds(row0, rpt)])


def _segment_sums(x, src, dst, n_pad):
    e_pad = src.shape[0]
    epw = e_pad // NW
    k_chunks = epw // C
    rpt = n_pad // NS
    d = x.shape[1]
    mesh = plsc.VectorSubcoreMesh(core_axis_name="c", subcore_axis_name="s")
    body = functools.partial(_sc_body, (n_pad, rpt, k_chunks, epw))
    return pl.kernel(
        body,
        out_type=(
            jax.ShapeDtypeStruct((NC, n_pad, d), jnp.float32),
            jax.ShapeDtypeStruct((NC, n_pad, 16), jnp.float32),
        ),
        mesh=mesh,
        compiler_params=pltpu.CompilerParams(use_tc_tiling_on_sc=False),
        scratch_types=[
            pltpu.VMEM((C,), jnp.int32),
            pltpu.VMEM((C,), jnp.int32),
            pltpu.VMEM((C, d), jnp.float32),
            pltpu.VMEM((C, 16), jnp.float32),
            pltpu.VMEM_SHARED((n_pad, d), jnp.float32),
            pltpu.VMEM_SHARED((n_pad, 16), jnp.float32),
            pltpu.SemaphoreType.DMA,
        ],
    )(x, src, dst)


def _tc_body(s_ref, c_ref, w_ref, b_ref, o_ref):
    s = s_ref[0] + s_ref[1]
    c = c_ref[0, :, 0:1] + c_ref[1, :, 0:1]
    mean = s / jnp.maximum(c, 1.0)
    h = lax.dot_general(mean, w_ref[...], (((1,), (1,)), ((), ())),
                        preferred_element_type=jnp.float32)
    out = h + b_ref[...] * jnp.minimum(c, 1.0)
    o_ref[...] = jnp.maximum(out, 0.0)


def _finish(sums, cnts, W, b, n_pad, rows_blk):
    d_in = W.shape[1]
    d_out = W.shape[0]
    grid = (n_pad // rows_blk,)
    return pl.pallas_call(
        _tc_body,
        grid=grid,
        in_specs=[
            pl.BlockSpec((NC, rows_blk, d_in), lambda i: (0, i, 0)),
            pl.BlockSpec((NC, rows_blk, 16), lambda i: (0, i, 0)),
            pl.BlockSpec((d_out, d_in), lambda i: (0, 0)),
            pl.BlockSpec((1, d_out), lambda i: (0, 0)),
        ],
        out_specs=pl.BlockSpec((rows_blk, d_out), lambda i: (i, 0)),
        out_shape=jax.ShapeDtypeStruct((n_pad, d_out), jnp.float32),
    )(sums, cnts, W, b.reshape(1, d_out))


def kernel(x, edge_index, W, b):
    n = x.shape[0]
    e = edge_index.shape[1]

    n_pad = ((n + 1) + NS * C - 1) // (NS * C) * (NS * C)
    e_pad = (e + NW * C - 1) // (NW * C) * (NW * C)

    src = edge_index[0].astype(jnp.int32)
    dst = edge_index[1].astype(jnp.int32)
    pad = e_pad - e
    if pad:
        src = jnp.concatenate([src, jnp.zeros((pad,), jnp.int32)])
        dst = jnp.concatenate([dst, jnp.full((pad,), n, jnp.int32)])

    sums, cnts = _segment_sums(x, src, dst, n_pad)
    out = _finish(sums, cnts, W, b, n_pad, rows_blk=1024)
    return out[:n]
